# baseline (speedup 1.0000x reference)
import functools

import jax
import jax.numpy as jnp
from jax import lax
from jax.experimental import pallas as pl
from jax.experimental.pallas import tpu as pltpu
from jax.experimental.pallas import tpu_sc as plsc

NUM_CORES = 2
NUM_SUBCORES = 16
LANES = 16
NW = NUM_CORES * NUM_SUBCORES
BATCH = 16384
EDIM = 32
B_PER_W = BATCH // NW
CHUNK = 128
NCHUNK = B_PER_W // CHUNK
GROUPS = B_PER_W // LANES


def _sc_pred_diff(u2, i2, j2, user_embed, item_embed):
    mesh = plsc.VectorSubcoreMesh(core_axis_name="c", subcore_axis_name="s")

    @functools.partial(
        pl.kernel,
        out_type=jax.ShapeDtypeStruct((BATCH,), jnp.float32),
        mesh=mesh,
        compiler_params=pltpu.CompilerParams(
            needs_layout_passes=False, use_tc_tiling_on_sc=False),
        scratch_types=[
            pltpu.VMEM((NCHUNK, CHUNK), jnp.int32),
            pltpu.VMEM((NCHUNK, CHUNK), jnp.int32),
            pltpu.VMEM((NCHUNK, CHUNK), jnp.int32),
            pltpu.VMEM((B_PER_W, EDIM), jnp.float32),
            pltpu.VMEM((B_PER_W, EDIM), jnp.float32),
            pltpu.VMEM((B_PER_W, EDIM), jnp.float32),
            pltpu.VMEM((B_PER_W,), jnp.float32),
            pltpu.SemaphoreType.DMA,
        ],
    )
    def run(u_hbm, i_hbm, j_hbm, ut_hbm, it_hbm, out_hbm,
            u_idx, i_idx, j_idx, ue_v, ie_v, je_v, pred_v, sem):
        wid = lax.axis_index("s") * NUM_CORES + lax.axis_index("c")
        row0 = wid * NCHUNK
        pltpu.sync_copy(u_hbm.at[pl.ds(row0, NCHUNK)], u_idx)
        pltpu.sync_copy(i_hbm.at[pl.ds(row0, NCHUNK)], i_idx)
        pltpu.sync_copy(j_hbm.at[pl.ds(row0, NCHUNK)], j_idx)
        copies = []
        for c in range(NCHUNK):
            dst = pl.ds(c * CHUNK, CHUNK)
            copies.append(pltpu.async_copy(ut_hbm.at[u_idx.at[c]], ue_v.at[dst], sem))
            copies.append(pltpu.async_copy(it_hbm.at[i_idx.at[c]], ie_v.at[dst], sem))
            copies.append(pltpu.async_copy(it_hbm.at[j_idx.at[c]], je_v.at[dst], sem))
        for cp in copies:
            cp.wait()

        lane = lax.iota(jnp.int32, LANES)

        def body(g, carry):
            base = pl.multiple_of(g * LANES, LANES)
            rows = base + lane
            acc = jnp.zeros((LANES,), jnp.float32)
            for dcol in range(EDIM):
                cols = jnp.full((LANES,), dcol, jnp.int32)
                uev = plsc.load_gather(ue_v, [rows, cols])
                iev = plsc.load_gather(ie_v, [rows, cols])
                jev = plsc.load_gather(je_v, [rows, cols])
                acc = acc + uev * (iev - jev)
            pred_v[pl.ds(base, LANES)] = acc
            return carry

        lax.fori_loop(0, GROUPS, body, 0)
        pltpu.sync_copy(pred_v, out_hbm.at[pl.ds(wid * B_PER_W, B_PER_W)])

    return run(u2, i2, j2, user_embed, item_embed)


def _tc_loss(dvals):
    x2 = dvals.reshape(CHUNK, CHUNK)

    def body(x_ref, o_ref):
        x = x_ref[...]
        ls = jnp.minimum(x, 0.0) - jnp.log(1.0 + jnp.exp(-jnp.abs(x)))
        o_ref[0, 0] = -jnp.sum(ls)

    out = pl.pallas_call(
        body,
        out_shape=jax.ShapeDtypeStruct((1, 1), jnp.float32),
        out_specs=pl.BlockSpec(memory_space=pltpu.SMEM),
    )(x2)
    return out[0, 0]


def kernel(u, i, j, user_embed, item_embed):
    u2 = u.astype(jnp.int32).reshape(NW * NCHUNK, CHUNK)
    i2 = i.astype(jnp.int32).reshape(NW * NCHUNK, CHUNK)
    j2 = j.astype(jnp.int32).reshape(NW * NCHUNK, CHUNK)
    d = _sc_pred_diff(u2, i2, j2, user_embed, item_embed)
    return _tc_loss(d)
